# baseline (speedup 1.0000x reference)
import jax
import jax.numpy as jnp
from jax.experimental import pallas as pl
from jax.experimental.pallas import tpu as pltpu

_NC = 16
_NB = 8
_LAG = 4
_ROWS = 100000 // _NC


def _relay_kernel(u_ref, i_ref, c_ref, ou_ref, oi_ref, oc_ref,
                  buf_u, buf_i, in_u, in_i, out_u, out_i, csem):
    def in_cp(src, buf, sems, k):
        return pltpu.make_async_copy(
            src.at[pl.ds(k * _ROWS, _ROWS)], buf.at[k % _NB], sems.at[k % _NB])

    def out_cp(dst, buf, sems, j):
        return pltpu.make_async_copy(
            buf.at[j % _NB], dst.at[pl.ds(j * _ROWS, _ROWS)], sems.at[j % _NB])

    streams = ((u_ref, ou_ref, buf_u, in_u, out_u),
               (i_ref, oi_ref, buf_i, in_i, out_i))

    pltpu.make_async_copy(c_ref, oc_ref, csem).start()
    for k in range(_NC + _LAG):
        for src, dst, buf, isem, osem in streams:
            if k < _NC:
                if k >= _NB:
                    out_cp(dst, buf, osem, k - _NB).wait()
                in_cp(src, buf, isem, k).start()
            j = k - _LAG
            if 0 <= j < _NC:
                in_cp(src, buf, isem, j).wait()
                out_cp(dst, buf, osem, j).start()
    for src, dst, buf, isem, osem in streams:
        for j in range(_NC - _NB, _NC):
            out_cp(dst, buf, osem, j).wait()
    pltpu.make_async_copy(c_ref, oc_ref, csem).wait()


def kernel(emb_user, emb_item, emb_category):
    n, d = emb_user.shape
    outs = pl.pallas_call(
        _relay_kernel,
        out_shape=tuple(
            jax.ShapeDtypeStruct(x.shape, x.dtype)
            for x in (emb_user, emb_item, emb_category)
        ),
        in_specs=[pl.BlockSpec(memory_space=pl.ANY)] * 3,
        out_specs=[pl.BlockSpec(memory_space=pl.ANY)] * 3,
        scratch_shapes=[
            pltpu.VMEM((_NB, _ROWS, d), jnp.float32),
            pltpu.VMEM((_NB, _ROWS, d), jnp.float32),
            pltpu.SemaphoreType.DMA((_NB,)),
            pltpu.SemaphoreType.DMA((_NB,)),
            pltpu.SemaphoreType.DMA((_NB,)),
            pltpu.SemaphoreType.DMA((_NB,)),
            pltpu.SemaphoreType.DMA,
        ],
    )(emb_user, emb_item, emb_category)
    return outs
